# baseline (speedup 1.0000x reference)
import jax
import jax.numpy as jnp
from jax import lax
from jax.experimental import pallas as pl
from jax.experimental.pallas import tpu as pltpu
from jax.experimental.pallas import tpu_sc as plsc

EPS = 1e-10
V = 50000
E = 10000
NI = 800000
S = 4
D = 16
SD = S * D
TAU = 1.0

NC = 2
NS = 16
NW = NC * NS

CH = 125
RPB = 8
NROW = NI // CH
TROWS = NROW // NW
NBLK = TROWS // RPB
NBUF = 6
NGRP = TROWS // NBUF

VH = V // 2
XROWS = VH + 88
XPW = XROWS // NS
YW = 1000
VW = 5000

_MESH = dict(core_axis_name="c", subcore_axis_name="s")


def _y_pass_kernel():
    mesh = plsc.VectorSubcoreMesh(**_MESH)
    out_type = jax.ShapeDtypeStruct((NC, E, SD), jnp.float32)
    scratch = ([
        pltpu.VMEM_SHARED((E, SD), jnp.float32),
        pltpu.VMEM((RPB, CH), jnp.int32),
        pltpu.VMEM((RPB, CH), jnp.int32),
    ] + [pltpu.VMEM((CH, SD), jnp.float32) for _ in range(NBUF)]
      + [pltpu.SemaphoreType.DMA for _ in range(NBUF)])

    def body(x_hbm, vidx, eidx, z64, ysum, yacc, vblk, eblk, *rest):
        bufs, sems = rest[:NBUF], rest[NBUF:]
        c = lax.axis_index("c")
        t = lax.axis_index("s")
        base = (t * NC + c) * TROWS

        @pl.when(t < 10)
        def _():
            pltpu.sync_copy(z64.at[pl.ds(0, YW)], yacc.at[pl.ds(t * YW, YW)])
        plsc.subcore_barrier()

        def blk(b, carry):
            r0 = base + b * RPB
            pltpu.sync_copy(vidx.at[pl.ds(r0, RPB)], vblk)
            pltpu.sync_copy(eidx.at[pl.ds(r0, RPB)], eblk)
            for j in range(NBUF):
                pltpu.async_copy(x_hbm.at[vblk.at[j]], bufs[j], sems[j])
            for j in range(RPB):
                jb = j % NBUF
                pltpu.make_async_copy(x_hbm.at[vblk.at[j]], bufs[jb],
                                      sems[jb]).wait()
                pltpu.sync_copy(bufs[jb], yacc.at[eblk.at[j]], add=True)
                if j + NBUF < RPB:
                    pltpu.async_copy(x_hbm.at[vblk.at[j + NBUF]], bufs[jb],
                                     sems[jb])
            return carry
        lax.fori_loop(0, NBLK, blk, 0)
        plsc.subcore_barrier()

        @pl.when(t < 10)
        def _():
            sl = pl.ds(t * YW, YW)
            pltpu.sync_copy(yacc.at[sl], ysum.at[c, sl])

    return pl.kernel(body, out_type=out_type, mesh=mesh, scratch_types=scratch,
                     compiler_params=pltpu.CompilerParams(use_tc_tiling_on_sc=False))


def _counts_kernel():
    mesh = plsc.VectorSubcoreMesh(**_MESH)
    out_type = (jax.ShapeDtypeStruct((NC, E, 16), jnp.float32),
                jax.ShapeDtypeStruct((NC, V, 16), jnp.float32))
    scratch = ([
        pltpu.VMEM_SHARED((E, 16), jnp.float32),
        pltpu.VMEM_SHARED((V, 16), jnp.float32),
        pltpu.VMEM((RPB, CH), jnp.int32),
        pltpu.VMEM((RPB, CH), jnp.int32),
        pltpu.VMEM((CH, 16), jnp.float32),
    ] + [pltpu.SemaphoreType.DMA for _ in range(2 * NBUF)])

    def body(vidx, eidx, z16, ones, ce_out, cv_out, ce, cv, vblk, eblk, onev,
             *sems):
        c = lax.axis_index("c")
        t = lax.axis_index("s")
        base = (t * NC + c) * TROWS
        pltpu.sync_copy(ones, onev)

        @pl.when(t < 10)
        def _():
            pltpu.sync_copy(z16.at[pl.ds(0, YW)], ce.at[pl.ds(t * YW, YW)])
            pltpu.sync_copy(z16, cv.at[pl.ds(t * VW, VW)])
        plsc.subcore_barrier()

        def blk(b, carry):
            r0 = base + b * RPB
            pltpu.sync_copy(vidx.at[pl.ds(r0, RPB)], vblk)
            pltpu.sync_copy(eidx.at[pl.ds(r0, RPB)], eblk)
            for j in range(RPB):
                jb = j % NBUF
                if j >= NBUF:
                    pltpu.make_async_copy(onev, ce.at[eblk.at[j - NBUF]],
                                          sems[jb]).wait()
                    pltpu.make_async_copy(onev, cv.at[vblk.at[j - NBUF]],
                                          sems[NBUF + jb]).wait()
                pltpu.async_copy(onev, ce.at[eblk.at[j]], sems[jb], add=True)
                pltpu.async_copy(onev, cv.at[vblk.at[j]], sems[NBUF + jb],
                                 add=True)
            for j in range(RPB - NBUF, RPB):
                jb = j % NBUF
                pltpu.make_async_copy(onev, ce.at[eblk.at[j]], sems[jb]).wait()
                pltpu.make_async_copy(onev, cv.at[vblk.at[j]],
                                      sems[NBUF + jb]).wait()
            return carry
        lax.fori_loop(0, NBLK, blk, 0)
        plsc.subcore_barrier()

        @pl.when(t < 10)
        def _():
            sl = pl.ds(t * YW, YW)
            pltpu.sync_copy(ce.at[sl], ce_out.at[c, sl])
            slv = pl.ds(t * VW, VW)
            pltpu.sync_copy(cv.at[slv], cv_out.at[c, slv])

    return pl.kernel(body, out_type=out_type, mesh=mesh, scratch_types=scratch,
                     compiler_params=pltpu.CompilerParams(use_tc_tiling_on_sc=False))


def _x_pass_kernel():
    NBX = 3
    TRX = NROW // NS
    NBLKX = TRX // RPB
    mesh = plsc.VectorSubcoreMesh(**_MESH)
    out_type = jax.ShapeDtypeStruct((NC, XROWS, SD), jnp.float32)
    scratch = ([
        pltpu.VMEM_SHARED((XROWS, SD), jnp.float32),
        pltpu.VMEM((RPB, CH), jnp.int32),
        pltpu.VMEM((RPB, CH), jnp.int32),
    ] + [pltpu.VMEM((CH, SD), jnp.float32) for _ in range(NBX)]
      + [pltpu.SemaphoreType.DMA for _ in range(NBX)])

    def body(y_hbm, eidx, vsb, z64, xsum, xacc, eblk, vblk, *rest):
        bufs, sems = rest[:NBX], rest[NBX:]
        c = lax.axis_index("c")
        t = lax.axis_index("s")
        base = t * TRX
        pltpu.sync_copy(z64, xacc.at[pl.ds(t * XPW, XPW)])
        plsc.subcore_barrier()

        def blk(b, carry):
            r0 = base + b * RPB
            pltpu.sync_copy(eidx.at[pl.ds(r0, RPB)], eblk)
            pltpu.sync_copy(vsb.at[c, pl.ds(r0, RPB)], vblk)
            for j in range(NBX):
                pltpu.async_copy(y_hbm.at[eblk.at[j]], bufs[j], sems[j])
            for j in range(RPB):
                jb = j % NBX
                pltpu.make_async_copy(y_hbm.at[eblk.at[j]], bufs[jb],
                                      sems[jb]).wait()
                pltpu.sync_copy(bufs[jb], xacc.at[vblk.at[j]], add=True)
                if j + NBX < RPB:
                    pltpu.async_copy(y_hbm.at[eblk.at[j + NBX]], bufs[jb],
                                     sems[jb])
            return carry
        lax.fori_loop(0, NBLKX, blk, 0)
        plsc.subcore_barrier()

        sl = pl.ds(t * XPW, XPW)
        pltpu.sync_copy(xacc.at[sl], xsum.at[c, sl])

    return pl.kernel(body, out_type=out_type, mesh=mesh, scratch_types=scratch,
                     compiler_params=pltpu.CompilerParams(use_tc_tiling_on_sc=False))


def _run_counts(vidx, eidx, z16, ones):
    return _counts_kernel()(vidx, eidx, z16, ones)


def _run_y(x, vidx, eidx, z64):
    return _y_pass_kernel()(x, vidx, eidx, z64)


def _run_x(y, eidx, vsb, z64):
    return _x_pass_kernel()(y, eidx, vsb, z64)


def _gs_body(emb_ref, g_ref, out_ref):
    x = (emb_ref[...] + g_ref[...]) / TAU
    ex = jnp.exp(x)
    gi = lax.broadcasted_iota(jnp.int32, (SD, SD), 0) // D
    gj = lax.broadcasted_iota(jnp.int32, (SD, SD), 1) // D
    bd = (gi == gj).astype(jnp.float32)
    s = lax.dot_general(ex, bd, (((1,), (0,)), ((), ())),
                        preferred_element_type=jnp.float32)
    out_ref[...] = ex / s


def _gumbel_softmax(emb, g):
    br = 1000
    return pl.pallas_call(
        _gs_body,
        grid=(V // br,),
        in_specs=[pl.BlockSpec((br, SD), lambda i: (i, 0)),
                  pl.BlockSpec((br, SD), lambda i: (i, 0))],
        out_specs=pl.BlockSpec((br, SD), lambda i: (i, 0)),
        out_shape=jax.ShapeDtypeStruct((V, SD), jnp.float32),
    )(emb, g)


def _stats_accumulate(y, ent_s, col_s, g_s):
    ent_s[...] = ent_s[...] + (-jnp.sum(y * jnp.log(y + EPS)))
    col_s[...] = col_s[...] + jnp.sum(y, axis=0)[None, :]
    g_s[...] = g_s[...] + lax.dot_general(y, y, (((0,), (0,)), ((), ())),
                                          preferred_element_type=jnp.float32)


def _stats_final(n, ent_s, col_s, g_s):
    local = jnp.sum(ent_s[...]) / (n * S)
    pcol = col_s[...] / n
    gb = jnp.sum(pcol * jnp.log(pcol + EPS)) / S
    g = g_s[...]
    eye = (lax.broadcasted_iota(jnp.int32, (D, D), 0) ==
           lax.broadcasted_iota(jnp.int32, (D, D), 1)).astype(jnp.float32)
    disc = jnp.float32(0.0)
    for s in range(S):
        gs = g[s * D:(s + 1) * D, s * D:(s + 1) * D]
        dg = jnp.sum(gs * eye, axis=1)
        norms = jnp.sqrt(dg)
        denom = jnp.maximum(norms[:, None] * norms[None, :], EPS)
        cs = gs / denom
        m = jnp.max(cs, axis=1, keepdims=True)
        ex = jnp.exp(cs - m)
        smd = jnp.sum(ex * eye, axis=1) / jnp.sum(ex, axis=1)
        disc = disc + jnp.sum(-jnp.log(smd))
    disc = disc / (S * D)
    return jnp.stack([local, gb + disc]).reshape(1, 2)


_STATS_SCRATCH = lambda: [pltpu.VMEM((1, 1), jnp.float32),
                          pltpu.VMEM((1, SD), jnp.float32),
                          pltpu.VMEM((SD, SD), jnp.float32)]


def _y_norm_kernel(first):
    br = 1000
    nb = E // br
    in_specs = [pl.BlockSpec((NC, br, SD), lambda k: (0, k, 0))]
    if first:
        in_specs.append(pl.BlockSpec((NC, br, 16), lambda k: (0, k, 0)))
    else:
        in_specs.append(pl.BlockSpec((br, 1), lambda k: (k, 0)))
    out_shape = [jax.ShapeDtypeStruct((E, SD), jnp.float32),
                 jax.ShapeDtypeStruct((1, 2), jnp.float32)]
    out_specs = [pl.BlockSpec((br, SD), lambda k: (k, 0)),
                 pl.BlockSpec((1, 2), lambda k: (0, 0))]
    if first:
        out_shape.append(jax.ShapeDtypeStruct((E, 1), jnp.float32))
        out_specs.append(pl.BlockSpec((br, 1), lambda k: (k, 0)))

    def body(ys_ref, cnt_ref, y_out, st_out, *rest):
        if first:
            cnt_out, ent_s, col_s, g_s = rest
        else:
            ent_s, col_s, g_s = rest
        k = pl.program_id(0)
        a = ys_ref[...]
        p = a[0] + a[1]
        if first:
            cb = cnt_ref[...]
            cnt = cb[0, :, :1] + cb[1, :, :1]
            cnt_out[...] = cnt
        else:
            cnt = cnt_ref[...]
        y = p * (1.0 / jnp.maximum(cnt, 1.0))
        y_out[...] = y

        @pl.when(k == 0)
        def _():
            ent_s[...] = jnp.zeros_like(ent_s)
            col_s[...] = jnp.zeros_like(col_s)
            g_s[...] = jnp.zeros_like(g_s)

        _stats_accumulate(y, ent_s, col_s, g_s)

        @pl.when(k == nb - 1)
        def _():
            st_out[...] = _stats_final(E, ent_s, col_s, g_s)

    return pl.pallas_call(body, grid=(nb,), in_specs=in_specs,
                          out_specs=out_specs, out_shape=out_shape,
                          scratch_shapes=_STATS_SCRATCH())


def _x_norm_kernel(first):
    br = 1000
    nb = VH // br
    in_specs = [pl.BlockSpec((1, br, SD), lambda h, k: (h, k, 0))]
    if first:
        in_specs.append(pl.BlockSpec((NC, br, 16), lambda h, k: (0, h * nb + k, 0)))
        out_shape = [jax.ShapeDtypeStruct((V, SD), jnp.float32),
                     jax.ShapeDtypeStruct((V, 1), jnp.float32)]
        out_specs = [pl.BlockSpec((br, SD), lambda h, k: (h * nb + k, 0)),
                     pl.BlockSpec((br, 1), lambda h, k: (h * nb + k, 0))]
        scratch = []
    else:
        in_specs.append(pl.BlockSpec((br, 1), lambda h, k: (h * nb + k, 0)))
        out_shape = [jax.ShapeDtypeStruct((1, 2), jnp.float32)]
        out_specs = [pl.BlockSpec((1, 2), lambda h, k: (0, 0))]
        scratch = _STATS_SCRATCH()

    def body(xs_ref, cnt_ref, *rest):
        h = pl.program_id(0)
        k = pl.program_id(1)
        p = xs_ref[0]
        if first:
            x_out, cnt_out = rest
            cb = cnt_ref[...]
            cnt = cb[0, :, :1] + cb[1, :, :1]
            cnt_out[...] = cnt
            x_out[...] = p * (1.0 / jnp.maximum(cnt, 1.0))
        else:
            st_out, ent_s, col_s, g_s = rest
            x = p * (1.0 / jnp.maximum(cnt_ref[...], 1.0))

            @pl.when((h == 0) & (k == 0))
            def _():
                ent_s[...] = jnp.zeros_like(ent_s)
                col_s[...] = jnp.zeros_like(col_s)
                g_s[...] = jnp.zeros_like(g_s)

            _stats_accumulate(x, ent_s, col_s, g_s)

            @pl.when((h == 1) & (k == nb - 1))
            def _():
                st_out[...] = _stats_final(V, ent_s, col_s, g_s)

    outs = out_shape if len(out_shape) > 1 else out_shape[0]
    return pl.pallas_call(body, grid=(2, nb), in_specs=in_specs,
                          out_specs=out_specs if len(out_shape) > 1 else out_specs[0],
                          out_shape=outs, scratch_shapes=scratch)


def _x_stats_kernel():
    br = 1000
    nb = V // br

    def body(x_ref, st_out, ent_s, col_s, g_s):
        k = pl.program_id(0)
        x = x_ref[...]

        @pl.when(k == 0)
        def _():
            ent_s[...] = jnp.zeros_like(ent_s)
            col_s[...] = jnp.zeros_like(col_s)
            g_s[...] = jnp.zeros_like(g_s)

        _stats_accumulate(x, ent_s, col_s, g_s)

        @pl.when(k == nb - 1)
        def _():
            st_out[...] = _stats_final(V, ent_s, col_s, g_s)

    return pl.pallas_call(
        body, grid=(nb,),
        in_specs=[pl.BlockSpec((br, SD), lambda k: (k, 0))],
        out_specs=pl.BlockSpec((1, 2), lambda k: (0, 0)),
        out_shape=jax.ShapeDtypeStruct((1, 2), jnp.float32),
        scratch_shapes=_STATS_SCRATCH())


def kernel(V_idx, E_idx, node_embedding):
    with jax.ensure_compile_time_eval():
        g = jax.random.gumbel(jax.random.key(42), (V, S, D),
                              dtype=jnp.float32).reshape(V, SD)
    vi = V_idx.astype(jnp.int32)
    ei = E_idx.astype(jnp.int32)
    vidx2 = vi.reshape(NROW, CH)
    eidx2 = ei.reshape(NROW, CH)
    spread = VH + (jnp.arange(NI, dtype=jnp.int32) % 64)
    vs0 = jnp.where(vi < VH, vi, spread).reshape(NROW, CH)
    vs1 = jnp.where(vi >= VH, vi - VH, spread).reshape(NROW, CH)
    vsb = jnp.stack([vs0, vs1])
    z64 = jnp.zeros((XPW, SD), jnp.float32)
    z16 = jnp.zeros((VW, 16), jnp.float32)
    ones16 = jnp.ones((CH, 16), jnp.float32)

    cep, cvp = _run_counts(vidx2, eidx2, z16, ones16)
    z64 = z64 + cep[0, 0, 0] * 0.0
    x = _gumbel_softmax(node_embedding, g)
    locs, globs = [], []
    cnt_e = cnt_v = None
    for step in range(2):
        ysum = _run_y(x, vidx2, eidx2, z64)
        if step == 0:
            y, st, cnt_e = _y_norm_kernel(True)(ysum, cep)
        else:
            y, st = _y_norm_kernel(False)(ysum, cnt_e)
        locs.append(st[0, 0])
        globs.append(st[0, 1])
        xsum = _run_x(y, eidx2, vsb, z64)
        if step == 0:
            x, cnt_v = _x_norm_kernel(True)(xsum, cvp)
            st = _x_stats_kernel()(x)
        else:
            st = _x_norm_kernel(False)(xsum, cnt_v)
        locs.append(st[0, 0])
        globs.append(st[0, 1])
    return jnp.stack([jnp.stack(locs), jnp.stack(globs)])


# speedup vs baseline: 1.0067x; 1.0067x over previous
import jax
import jax.numpy as jnp
from jax import lax
from jax.experimental import pallas as pl
from jax.experimental.pallas import tpu as pltpu
from jax.experimental.pallas import tpu_sc as plsc

EPS = 1e-10
V = 50000
E = 10000
NI = 800000
S = 4
D = 16
SD = S * D
TAU = 1.0

NC = 2
NS = 16
NW = NC * NS

CH = 125
RPB = 8
NROW = NI // CH
TROWS = NROW // NW
NBLK = TROWS // RPB
NBUF = 4
NGRP = TROWS // NBUF

VH = V // 2
XROWS = VH + 88
XPW = XROWS // NS
YW = 1000
VW = 5000

_MESH = dict(core_axis_name="c", subcore_axis_name="s")


def _y_pass_kernel():
    mesh = plsc.VectorSubcoreMesh(**_MESH)
    out_type = jax.ShapeDtypeStruct((NC, E, SD), jnp.float32)
    scratch = ([
        pltpu.VMEM_SHARED((E, SD), jnp.float32),
        pltpu.VMEM((RPB, CH), jnp.int32),
        pltpu.VMEM((RPB, CH), jnp.int32),
    ] + [pltpu.VMEM((CH, SD), jnp.float32) for _ in range(NBUF)]
      + [pltpu.SemaphoreType.DMA for _ in range(NBUF)])

    def body(x_hbm, vidx, eidx, z64, ysum, yacc, vblk, eblk, *rest):
        bufs, sems = rest[:NBUF], rest[NBUF:]
        c = lax.axis_index("c")
        t = lax.axis_index("s")
        base = (t * NC + c) * TROWS

        @pl.when(t < 10)
        def _():
            pltpu.sync_copy(z64.at[pl.ds(0, YW)], yacc.at[pl.ds(t * YW, YW)])
        plsc.subcore_barrier()

        def blk(b, carry):
            r0 = base + b * RPB
            pltpu.sync_copy(vidx.at[pl.ds(r0, RPB)], vblk)
            pltpu.sync_copy(eidx.at[pl.ds(r0, RPB)], eblk)
            for j in range(NBUF):
                pltpu.async_copy(x_hbm.at[vblk.at[j]], bufs[j], sems[j])
            for j in range(RPB):
                jb = j % NBUF
                pltpu.make_async_copy(x_hbm.at[vblk.at[j]], bufs[jb],
                                      sems[jb]).wait()
                pltpu.sync_copy(bufs[jb], yacc.at[eblk.at[j]], add=True)
                if j + NBUF < RPB:
                    pltpu.async_copy(x_hbm.at[vblk.at[j + NBUF]], bufs[jb],
                                     sems[jb])
            return carry
        lax.fori_loop(0, NBLK, blk, 0)
        plsc.subcore_barrier()

        @pl.when(t < 10)
        def _():
            sl = pl.ds(t * YW, YW)
            pltpu.sync_copy(yacc.at[sl], ysum.at[c, sl])

    return pl.kernel(body, out_type=out_type, mesh=mesh, scratch_types=scratch,
                     compiler_params=pltpu.CompilerParams(use_tc_tiling_on_sc=False))


def _counts_kernel():
    mesh = plsc.VectorSubcoreMesh(**_MESH)
    out_type = (jax.ShapeDtypeStruct((NC, E, 16), jnp.float32),
                jax.ShapeDtypeStruct((NC, V, 16), jnp.float32))
    scratch = ([
        pltpu.VMEM_SHARED((E, 16), jnp.float32),
        pltpu.VMEM_SHARED((V, 16), jnp.float32),
        pltpu.VMEM((RPB, CH), jnp.int32),
        pltpu.VMEM((RPB, CH), jnp.int32),
        pltpu.VMEM((CH, 16), jnp.float32),
    ] + [pltpu.SemaphoreType.DMA for _ in range(2 * NBUF)])

    def body(vidx, eidx, z16, ones, ce_out, cv_out, ce, cv, vblk, eblk, onev,
             *sems):
        c = lax.axis_index("c")
        t = lax.axis_index("s")
        base = (t * NC + c) * TROWS
        pltpu.sync_copy(ones, onev)

        @pl.when(t < 10)
        def _():
            pltpu.sync_copy(z16.at[pl.ds(0, YW)], ce.at[pl.ds(t * YW, YW)])
            pltpu.sync_copy(z16, cv.at[pl.ds(t * VW, VW)])
        plsc.subcore_barrier()

        def blk(b, carry):
            r0 = base + b * RPB
            pltpu.sync_copy(vidx.at[pl.ds(r0, RPB)], vblk)
            pltpu.sync_copy(eidx.at[pl.ds(r0, RPB)], eblk)
            for j in range(RPB):
                jb = j % NBUF
                if j >= NBUF:
                    pltpu.make_async_copy(onev, ce.at[eblk.at[j - NBUF]],
                                          sems[jb]).wait()
                    pltpu.make_async_copy(onev, cv.at[vblk.at[j - NBUF]],
                                          sems[NBUF + jb]).wait()
                pltpu.async_copy(onev, ce.at[eblk.at[j]], sems[jb], add=True)
                pltpu.async_copy(onev, cv.at[vblk.at[j]], sems[NBUF + jb],
                                 add=True)
            for j in range(RPB - NBUF, RPB):
                jb = j % NBUF
                pltpu.make_async_copy(onev, ce.at[eblk.at[j]], sems[jb]).wait()
                pltpu.make_async_copy(onev, cv.at[vblk.at[j]],
                                      sems[NBUF + jb]).wait()
            return carry
        lax.fori_loop(0, NBLK, blk, 0)
        plsc.subcore_barrier()

        @pl.when(t < 10)
        def _():
            sl = pl.ds(t * YW, YW)
            pltpu.sync_copy(ce.at[sl], ce_out.at[c, sl])
            slv = pl.ds(t * VW, VW)
            pltpu.sync_copy(cv.at[slv], cv_out.at[c, slv])

    return pl.kernel(body, out_type=out_type, mesh=mesh, scratch_types=scratch,
                     compiler_params=pltpu.CompilerParams(use_tc_tiling_on_sc=False))


def _x_pass_kernel():
    NBX = 3
    TRX = NROW // NS
    NBLKX = TRX // RPB
    mesh = plsc.VectorSubcoreMesh(**_MESH)
    out_type = jax.ShapeDtypeStruct((NC, XROWS, SD), jnp.float32)
    scratch = ([
        pltpu.VMEM_SHARED((XROWS, SD), jnp.float32),
        pltpu.VMEM((RPB, CH), jnp.int32),
        pltpu.VMEM((RPB, CH), jnp.int32),
    ] + [pltpu.VMEM((CH, SD), jnp.float32) for _ in range(NBX)]
      + [pltpu.SemaphoreType.DMA for _ in range(NBX)])

    def body(y_hbm, eidx, vsb, z64, xsum, xacc, eblk, vblk, *rest):
        bufs, sems = rest[:NBX], rest[NBX:]
        c = lax.axis_index("c")
        t = lax.axis_index("s")
        base = t * TRX
        pltpu.sync_copy(z64, xacc.at[pl.ds(t * XPW, XPW)])
        plsc.subcore_barrier()

        def blk(b, carry):
            r0 = base + b * RPB
            pltpu.sync_copy(eidx.at[pl.ds(r0, RPB)], eblk)
            pltpu.sync_copy(vsb.at[c, pl.ds(r0, RPB)], vblk)
            for j in range(NBX):
                pltpu.async_copy(y_hbm.at[eblk.at[j]], bufs[j], sems[j])
            for j in range(RPB):
                jb = j % NBX
                pltpu.make_async_copy(y_hbm.at[eblk.at[j]], bufs[jb],
                                      sems[jb]).wait()
                pltpu.sync_copy(bufs[jb], xacc.at[vblk.at[j]], add=True)
                if j + NBX < RPB:
                    pltpu.async_copy(y_hbm.at[eblk.at[j + NBX]], bufs[jb],
                                     sems[jb])
            return carry
        lax.fori_loop(0, NBLKX, blk, 0)
        plsc.subcore_barrier()

        sl = pl.ds(t * XPW, XPW)
        pltpu.sync_copy(xacc.at[sl], xsum.at[c, sl])

    return pl.kernel(body, out_type=out_type, mesh=mesh, scratch_types=scratch,
                     compiler_params=pltpu.CompilerParams(use_tc_tiling_on_sc=False))


def _run_counts(vidx, eidx, z16, ones):
    return _counts_kernel()(vidx, eidx, z16, ones)


def _run_y(x, vidx, eidx, z64):
    return _y_pass_kernel()(x, vidx, eidx, z64)


def _run_x(y, eidx, vsb, z64):
    return _x_pass_kernel()(y, eidx, vsb, z64)


def _gs_body(emb_ref, g_ref, out_ref):
    x = (emb_ref[...] + g_ref[...]) / TAU
    ex = jnp.exp(x)
    gi = lax.broadcasted_iota(jnp.int32, (SD, SD), 0) // D
    gj = lax.broadcasted_iota(jnp.int32, (SD, SD), 1) // D
    bd = (gi == gj).astype(jnp.float32)
    s = lax.dot_general(ex, bd, (((1,), (0,)), ((), ())),
                        preferred_element_type=jnp.float32)
    out_ref[...] = ex / s


def _gumbel_softmax(emb, g):
    br = 1000
    return pl.pallas_call(
        _gs_body,
        grid=(V // br,),
        in_specs=[pl.BlockSpec((br, SD), lambda i: (i, 0)),
                  pl.BlockSpec((br, SD), lambda i: (i, 0))],
        out_specs=pl.BlockSpec((br, SD), lambda i: (i, 0)),
        out_shape=jax.ShapeDtypeStruct((V, SD), jnp.float32),
    )(emb, g)


def _stats_accumulate(y, ent_s, col_s, g_s):
    ent_s[...] = ent_s[...] + (-jnp.sum(y * jnp.log(y + EPS)))
    col_s[...] = col_s[...] + jnp.sum(y, axis=0)[None, :]
    g_s[...] = g_s[...] + lax.dot_general(y, y, (((0,), (0,)), ((), ())),
                                          preferred_element_type=jnp.float32)


def _stats_final(n, ent_s, col_s, g_s):
    local = jnp.sum(ent_s[...]) / (n * S)
    pcol = col_s[...] / n
    gb = jnp.sum(pcol * jnp.log(pcol + EPS)) / S
    g = g_s[...]
    eye = (lax.broadcasted_iota(jnp.int32, (D, D), 0) ==
           lax.broadcasted_iota(jnp.int32, (D, D), 1)).astype(jnp.float32)
    disc = jnp.float32(0.0)
    for s in range(S):
        gs = g[s * D:(s + 1) * D, s * D:(s + 1) * D]
        dg = jnp.sum(gs * eye, axis=1)
        norms = jnp.sqrt(dg)
        denom = jnp.maximum(norms[:, None] * norms[None, :], EPS)
        cs = gs / denom
        m = jnp.max(cs, axis=1, keepdims=True)
        ex = jnp.exp(cs - m)
        smd = jnp.sum(ex * eye, axis=1) / jnp.sum(ex, axis=1)
        disc = disc + jnp.sum(-jnp.log(smd))
    disc = disc / (S * D)
    return jnp.stack([local, gb + disc]).reshape(1, 2)


_STATS_SCRATCH = lambda: [pltpu.VMEM((1, 1), jnp.float32),
                          pltpu.VMEM((1, SD), jnp.float32),
                          pltpu.VMEM((SD, SD), jnp.float32)]


def _y_norm_kernel(first):
    br = 1000
    nb = E // br
    in_specs = [pl.BlockSpec((NC, br, SD), lambda k: (0, k, 0))]
    if first:
        in_specs.append(pl.BlockSpec((NC, br, 16), lambda k: (0, k, 0)))
    else:
        in_specs.append(pl.BlockSpec((br, 1), lambda k: (k, 0)))
    out_shape = [jax.ShapeDtypeStruct((E, SD), jnp.float32),
                 jax.ShapeDtypeStruct((1, 2), jnp.float32)]
    out_specs = [pl.BlockSpec((br, SD), lambda k: (k, 0)),
                 pl.BlockSpec((1, 2), lambda k: (0, 0))]
    if first:
        out_shape.append(jax.ShapeDtypeStruct((E, 1), jnp.float32))
        out_specs.append(pl.BlockSpec((br, 1), lambda k: (k, 0)))

    def body(ys_ref, cnt_ref, y_out, st_out, *rest):
        if first:
            cnt_out, ent_s, col_s, g_s = rest
        else:
            ent_s, col_s, g_s = rest
        k = pl.program_id(0)
        a = ys_ref[...]
        p = a[0] + a[1]
        if first:
            cb = cnt_ref[...]
            cnt = cb[0, :, :1] + cb[1, :, :1]
            cnt_out[...] = cnt
        else:
            cnt = cnt_ref[...]
        y = p * (1.0 / jnp.maximum(cnt, 1.0))
        y_out[...] = y

        @pl.when(k == 0)
        def _():
            ent_s[...] = jnp.zeros_like(ent_s)
            col_s[...] = jnp.zeros_like(col_s)
            g_s[...] = jnp.zeros_like(g_s)

        _stats_accumulate(y, ent_s, col_s, g_s)

        @pl.when(k == nb - 1)
        def _():
            st_out[...] = _stats_final(E, ent_s, col_s, g_s)

    return pl.pallas_call(body, grid=(nb,), in_specs=in_specs,
                          out_specs=out_specs, out_shape=out_shape,
                          scratch_shapes=_STATS_SCRATCH())


def _x_norm_kernel(first):
    br = 1000
    nb = VH // br
    in_specs = [pl.BlockSpec((1, br, SD), lambda h, k: (h, k, 0))]
    if first:
        in_specs.append(pl.BlockSpec((NC, br, 16), lambda h, k: (0, h * nb + k, 0)))
        out_shape = [jax.ShapeDtypeStruct((V, SD), jnp.float32),
                     jax.ShapeDtypeStruct((V, 1), jnp.float32)]
        out_specs = [pl.BlockSpec((br, SD), lambda h, k: (h * nb + k, 0)),
                     pl.BlockSpec((br, 1), lambda h, k: (h * nb + k, 0))]
        scratch = []
    else:
        in_specs.append(pl.BlockSpec((br, 1), lambda h, k: (h * nb + k, 0)))
        out_shape = [jax.ShapeDtypeStruct((1, 2), jnp.float32)]
        out_specs = [pl.BlockSpec((1, 2), lambda h, k: (0, 0))]
        scratch = _STATS_SCRATCH()

    def body(xs_ref, cnt_ref, *rest):
        h = pl.program_id(0)
        k = pl.program_id(1)
        p = xs_ref[0]
        if first:
            x_out, cnt_out = rest
            cb = cnt_ref[...]
            cnt = cb[0, :, :1] + cb[1, :, :1]
            cnt_out[...] = cnt
            x_out[...] = p * (1.0 / jnp.maximum(cnt, 1.0))
        else:
            st_out, ent_s, col_s, g_s = rest
            x = p * (1.0 / jnp.maximum(cnt_ref[...], 1.0))

            @pl.when((h == 0) & (k == 0))
            def _():
                ent_s[...] = jnp.zeros_like(ent_s)
                col_s[...] = jnp.zeros_like(col_s)
                g_s[...] = jnp.zeros_like(g_s)

            _stats_accumulate(x, ent_s, col_s, g_s)

            @pl.when((h == 1) & (k == nb - 1))
            def _():
                st_out[...] = _stats_final(V, ent_s, col_s, g_s)

    outs = out_shape if len(out_shape) > 1 else out_shape[0]
    return pl.pallas_call(body, grid=(2, nb), in_specs=in_specs,
                          out_specs=out_specs if len(out_shape) > 1 else out_specs[0],
                          out_shape=outs, scratch_shapes=scratch)


def _x_stats_kernel():
    br = 1000
    nb = V // br

    def body(x_ref, st_out, ent_s, col_s, g_s):
        k = pl.program_id(0)
        x = x_ref[...]

        @pl.when(k == 0)
        def _():
            ent_s[...] = jnp.zeros_like(ent_s)
            col_s[...] = jnp.zeros_like(col_s)
            g_s[...] = jnp.zeros_like(g_s)

        _stats_accumulate(x, ent_s, col_s, g_s)

        @pl.when(k == nb - 1)
        def _():
            st_out[...] = _stats_final(V, ent_s, col_s, g_s)

    return pl.pallas_call(
        body, grid=(nb,),
        in_specs=[pl.BlockSpec((br, SD), lambda k: (k, 0))],
        out_specs=pl.BlockSpec((1, 2), lambda k: (0, 0)),
        out_shape=jax.ShapeDtypeStruct((1, 2), jnp.float32),
        scratch_shapes=_STATS_SCRATCH())


def kernel(V_idx, E_idx, node_embedding):
    with jax.ensure_compile_time_eval():
        g = jax.random.gumbel(jax.random.key(42), (V, S, D),
                              dtype=jnp.float32).reshape(V, SD)
    vi = V_idx.astype(jnp.int32)
    ei = E_idx.astype(jnp.int32)
    vidx2 = vi.reshape(NROW, CH)
    eidx2 = ei.reshape(NROW, CH)
    spread = VH + (jnp.arange(NI, dtype=jnp.int32) % 64)
    vs0 = jnp.where(vi < VH, vi, spread).reshape(NROW, CH)
    vs1 = jnp.where(vi >= VH, vi - VH, spread).reshape(NROW, CH)
    vsb = jnp.stack([vs0, vs1])
    z64 = jnp.zeros((XPW, SD), jnp.float32)
    z16 = jnp.zeros((VW, 16), jnp.float32)
    ones16 = jnp.ones((CH, 16), jnp.float32)

    cep, cvp = _run_counts(vidx2, eidx2, z16, ones16)
    z64 = z64 + cep[0, 0, 0] * 0.0
    x = _gumbel_softmax(node_embedding, g)
    locs, globs = [], []
    cnt_e = cnt_v = None
    for step in range(2):
        ysum = _run_y(x, vidx2, eidx2, z64)
        if step == 0:
            y, st, cnt_e = _y_norm_kernel(True)(ysum, cep)
        else:
            y, st = _y_norm_kernel(False)(ysum, cnt_e)
        locs.append(st[0, 0])
        globs.append(st[0, 1])
        xsum = _run_x(y, eidx2, vsb, z64)
        if step == 0:
            x, cnt_v = _x_norm_kernel(True)(xsum, cvp)
            st = _x_stats_kernel()(x)
        else:
            st = _x_norm_kernel(False)(xsum, cnt_v)
        locs.append(st[0, 0])
        globs.append(st[0, 1])
    return jnp.stack([jnp.stack(locs), jnp.stack(globs)])


# speedup vs baseline: 1.0087x; 1.0019x over previous
import jax
import jax.numpy as jnp
from jax import lax
from jax.experimental import pallas as pl
from jax.experimental.pallas import tpu as pltpu
from jax.experimental.pallas import tpu_sc as plsc

EPS = 1e-10
V = 50000
E = 10000
NI = 800000
S = 4
D = 16
SD = S * D
TAU = 1.0

NC = 2
NS = 16
NW = NC * NS

CH = 125
RPB = 8
NROW = NI // CH
TROWS = NROW // NW
NBLK = TROWS // RPB
NBUF = 4
NGRP = TROWS // NBUF

VH = V // 2
XROWS = VH + 88
XPW = XROWS // NS
YW = 1000
VW = 5000

_MESH = dict(core_axis_name="c", subcore_axis_name="s")


def _y_pass_kernel():
    mesh = plsc.VectorSubcoreMesh(**_MESH)
    out_type = jax.ShapeDtypeStruct((NC, E, SD), jnp.float32)
    scratch = ([
        pltpu.VMEM_SHARED((E, SD), jnp.float32),
        pltpu.VMEM((RPB, CH), jnp.int32),
        pltpu.VMEM((RPB, CH), jnp.int32),
    ] + [pltpu.VMEM((CH, SD), jnp.float32) for _ in range(NBUF)]
      + [pltpu.SemaphoreType.DMA for _ in range(NBUF)])

    def body(x_hbm, vidx, eidx, z64, ysum, yacc, vblk, eblk, *rest):
        bufs, sems = rest[:NBUF], rest[NBUF:]
        c = lax.axis_index("c")
        t = lax.axis_index("s")
        base = (t * NC + c) * TROWS

        @pl.when(t < 10)
        def _():
            pltpu.sync_copy(z64.at[pl.ds(0, YW)], yacc.at[pl.ds(t * YW, YW)])
        plsc.subcore_barrier()

        def blk(b, carry):
            r0 = base + b * RPB
            pltpu.sync_copy(vidx.at[pl.ds(r0, RPB)], vblk)
            pltpu.sync_copy(eidx.at[pl.ds(r0, RPB)], eblk)
            for j in range(NBUF):
                pltpu.async_copy(x_hbm.at[vblk.at[j]], bufs[j], sems[j])
            for j in range(RPB):
                jb = j % NBUF
                pltpu.make_async_copy(x_hbm.at[vblk.at[j]], bufs[jb],
                                      sems[jb]).wait()
                pltpu.sync_copy(bufs[jb], yacc.at[eblk.at[j]], add=True)
                if j + NBUF < RPB:
                    pltpu.async_copy(x_hbm.at[vblk.at[j + NBUF]], bufs[jb],
                                     sems[jb])
            return carry
        lax.fori_loop(0, NBLK, blk, 0)
        plsc.subcore_barrier()

        @pl.when(t < 10)
        def _():
            sl = pl.ds(t * YW, YW)
            pltpu.sync_copy(yacc.at[sl], ysum.at[c, sl])

    return pl.kernel(body, out_type=out_type, mesh=mesh, scratch_types=scratch,
                     compiler_params=pltpu.CompilerParams(use_tc_tiling_on_sc=False))


def _counts_kernel():
    mesh = plsc.VectorSubcoreMesh(**_MESH)
    out_type = (jax.ShapeDtypeStruct((NC, E, 16), jnp.float32),
                jax.ShapeDtypeStruct((NC, V, 16), jnp.float32))
    scratch = ([
        pltpu.VMEM_SHARED((E, 16), jnp.float32),
        pltpu.VMEM_SHARED((V, 16), jnp.float32),
        pltpu.VMEM((RPB, CH), jnp.int32),
        pltpu.VMEM((RPB, CH), jnp.int32),
        pltpu.VMEM((CH, 16), jnp.float32),
    ] + [pltpu.SemaphoreType.DMA for _ in range(2 * NBUF)])

    def body(vidx, eidx, z16, ones, ce_out, cv_out, ce, cv, vblk, eblk, onev,
             *sems):
        c = lax.axis_index("c")
        t = lax.axis_index("s")
        base = (t * NC + c) * TROWS
        pltpu.sync_copy(ones, onev)

        @pl.when(t < 10)
        def _():
            pltpu.sync_copy(z16.at[pl.ds(0, YW)], ce.at[pl.ds(t * YW, YW)])
            pltpu.sync_copy(z16, cv.at[pl.ds(t * VW, VW)])
        plsc.subcore_barrier()

        def blk(b, carry):
            r0 = base + b * RPB
            pltpu.sync_copy(vidx.at[pl.ds(r0, RPB)], vblk)
            pltpu.sync_copy(eidx.at[pl.ds(r0, RPB)], eblk)
            for j in range(RPB):
                jb = j % NBUF
                if j >= NBUF:
                    pltpu.make_async_copy(onev, ce.at[eblk.at[j - NBUF]],
                                          sems[jb]).wait()
                    pltpu.make_async_copy(onev, cv.at[vblk.at[j - NBUF]],
                                          sems[NBUF + jb]).wait()
                pltpu.async_copy(onev, ce.at[eblk.at[j]], sems[jb], add=True)
                pltpu.async_copy(onev, cv.at[vblk.at[j]], sems[NBUF + jb],
                                 add=True)
            for j in range(RPB - NBUF, RPB):
                jb = j % NBUF
                pltpu.make_async_copy(onev, ce.at[eblk.at[j]], sems[jb]).wait()
                pltpu.make_async_copy(onev, cv.at[vblk.at[j]],
                                      sems[NBUF + jb]).wait()
            return carry
        lax.fori_loop(0, NBLK, blk, 0)
        plsc.subcore_barrier()

        @pl.when(t < 10)
        def _():
            sl = pl.ds(t * YW, YW)
            pltpu.sync_copy(ce.at[sl], ce_out.at[c, sl])
            slv = pl.ds(t * VW, VW)
            pltpu.sync_copy(cv.at[slv], cv_out.at[c, slv])

    return pl.kernel(body, out_type=out_type, mesh=mesh, scratch_types=scratch,
                     compiler_params=pltpu.CompilerParams(use_tc_tiling_on_sc=False))


def _x_pass_kernel():
    NBX = 3
    TRX = NROW // NS
    NBLKX = TRX // RPB
    mesh = plsc.VectorSubcoreMesh(**_MESH)
    out_type = jax.ShapeDtypeStruct((NC, XROWS, SD), jnp.float32)
    scratch = ([
        pltpu.VMEM_SHARED((XROWS, SD), jnp.float32),
        pltpu.VMEM((RPB, CH), jnp.int32),
        pltpu.VMEM((RPB, CH), jnp.int32),
    ] + [pltpu.VMEM((CH, SD), jnp.float32) for _ in range(NBX)]
      + [pltpu.SemaphoreType.DMA for _ in range(NBX)])

    def body(y_hbm, eidx, vsb, z64, xsum, xacc, eblk, vblk, *rest):
        bufs, sems = rest[:NBX], rest[NBX:]
        c = lax.axis_index("c")
        t = lax.axis_index("s")
        base = t * TRX
        pltpu.sync_copy(z64, xacc.at[pl.ds(t * XPW, XPW)])
        plsc.subcore_barrier()

        def blk(b, carry):
            r0 = base + b * RPB
            pltpu.sync_copy(eidx.at[pl.ds(r0, RPB)], eblk)
            pltpu.sync_copy(vsb.at[c, pl.ds(r0, RPB)], vblk)
            for j in range(NBX):
                pltpu.async_copy(y_hbm.at[eblk.at[j]], bufs[j], sems[j])
            for j in range(RPB):
                jb = j % NBX
                pltpu.make_async_copy(y_hbm.at[eblk.at[j]], bufs[jb],
                                      sems[jb]).wait()
                pltpu.sync_copy(bufs[jb], xacc.at[vblk.at[j]], add=True)
                if j + NBX < RPB:
                    pltpu.async_copy(y_hbm.at[eblk.at[j + NBX]], bufs[jb],
                                     sems[jb])
            return carry
        lax.fori_loop(0, NBLKX, blk, 0)
        plsc.subcore_barrier()

        sl = pl.ds(t * XPW, XPW)
        pltpu.sync_copy(xacc.at[sl], xsum.at[c, sl])

    return pl.kernel(body, out_type=out_type, mesh=mesh, scratch_types=scratch,
                     compiler_params=pltpu.CompilerParams(use_tc_tiling_on_sc=False))


def _run_counts(vidx, eidx, z16, ones):
    return _counts_kernel()(vidx, eidx, z16, ones)


def _run_y(x, vidx, eidx, z64):
    return _y_pass_kernel()(x, vidx, eidx, z64)


def _run_x(y, eidx, vsb, z64):
    return _x_pass_kernel()(y, eidx, vsb, z64)


def _gs_body(emb_ref, g_ref, out_ref):
    x = (emb_ref[...] + g_ref[...]) * (1.0 / TAU)
    ex = jnp.exp(x)
    gi = lax.broadcasted_iota(jnp.int32, (SD, SD), 0) // D
    gj = lax.broadcasted_iota(jnp.int32, (SD, SD), 1) // D
    bd = (gi == gj).astype(jnp.float32)
    s = lax.dot_general(ex, bd, (((1,), (0,)), ((), ())),
                        preferred_element_type=jnp.float32)
    out_ref[...] = ex / s


def _gumbel_softmax(emb, g):
    br = 2000
    return pl.pallas_call(
        _gs_body,
        grid=(V // br,),
        in_specs=[pl.BlockSpec((br, SD), lambda i: (i, 0)),
                  pl.BlockSpec((br, SD), lambda i: (i, 0))],
        out_specs=pl.BlockSpec((br, SD), lambda i: (i, 0)),
        out_shape=jax.ShapeDtypeStruct((V, SD), jnp.float32),
    )(emb, g)


def _stats_accumulate(y, ent_s, col_s, g_s):
    ent_s[...] = ent_s[...] + (-jnp.sum(y * jnp.log(y + EPS)))
    col_s[...] = col_s[...] + jnp.sum(y, axis=0)[None, :]
    g_s[...] = g_s[...] + lax.dot_general(y, y, (((0,), (0,)), ((), ())),
                                          preferred_element_type=jnp.float32)


def _stats_final(n, ent_s, col_s, g_s):
    local = jnp.sum(ent_s[...]) / (n * S)
    pcol = col_s[...] / n
    gb = jnp.sum(pcol * jnp.log(pcol + EPS)) / S
    g = g_s[...]
    eye = (lax.broadcasted_iota(jnp.int32, (D, D), 0) ==
           lax.broadcasted_iota(jnp.int32, (D, D), 1)).astype(jnp.float32)
    disc = jnp.float32(0.0)
    for s in range(S):
        gs = g[s * D:(s + 1) * D, s * D:(s + 1) * D]
        dg = jnp.sum(gs * eye, axis=1)
        norms = jnp.sqrt(dg)
        denom = jnp.maximum(norms[:, None] * norms[None, :], EPS)
        cs = gs / denom
        m = jnp.max(cs, axis=1, keepdims=True)
        ex = jnp.exp(cs - m)
        smd = jnp.sum(ex * eye, axis=1) / jnp.sum(ex, axis=1)
        disc = disc + jnp.sum(-jnp.log(smd))
    disc = disc / (S * D)
    return jnp.stack([local, gb + disc]).reshape(1, 2)


_STATS_SCRATCH = lambda: [pltpu.VMEM((1, 1), jnp.float32),
                          pltpu.VMEM((1, SD), jnp.float32),
                          pltpu.VMEM((SD, SD), jnp.float32)]


def _y_norm_kernel(first):
    br = 1000
    nb = E // br
    in_specs = [pl.BlockSpec((NC, br, SD), lambda k: (0, k, 0))]
    if first:
        in_specs.append(pl.BlockSpec((NC, br, 16), lambda k: (0, k, 0)))
    else:
        in_specs.append(pl.BlockSpec((br, 1), lambda k: (k, 0)))
    out_shape = [jax.ShapeDtypeStruct((E, SD), jnp.float32),
                 jax.ShapeDtypeStruct((1, 2), jnp.float32)]
    out_specs = [pl.BlockSpec((br, SD), lambda k: (k, 0)),
                 pl.BlockSpec((1, 2), lambda k: (0, 0))]
    if first:
        out_shape.append(jax.ShapeDtypeStruct((E, 1), jnp.float32))
        out_specs.append(pl.BlockSpec((br, 1), lambda k: (k, 0)))

    def body(ys_ref, cnt_ref, y_out, st_out, *rest):
        if first:
            cnt_out, ent_s, col_s, g_s = rest
        else:
            ent_s, col_s, g_s = rest
        k = pl.program_id(0)
        a = ys_ref[...]
        p = a[0] + a[1]
        if first:
            cb = cnt_ref[...]
            cnt = cb[0, :, :1] + cb[1, :, :1]
            cnt_out[...] = cnt
        else:
            cnt = cnt_ref[...]
        y = p * (1.0 / jnp.maximum(cnt, 1.0))
        y_out[...] = y

        @pl.when(k == 0)
        def _():
            ent_s[...] = jnp.zeros_like(ent_s)
            col_s[...] = jnp.zeros_like(col_s)
            g_s[...] = jnp.zeros_like(g_s)

        _stats_accumulate(y, ent_s, col_s, g_s)

        @pl.when(k == nb - 1)
        def _():
            st_out[...] = _stats_final(E, ent_s, col_s, g_s)

    return pl.pallas_call(body, grid=(nb,), in_specs=in_specs,
                          out_specs=out_specs, out_shape=out_shape,
                          scratch_shapes=_STATS_SCRATCH())


def _x_norm_kernel(first):
    br = 1000
    nb = VH // br
    in_specs = [pl.BlockSpec((1, br, SD), lambda h, k: (h, k, 0))]
    if first:
        in_specs.append(pl.BlockSpec((NC, br, 16), lambda h, k: (0, h * nb + k, 0)))
        out_shape = [jax.ShapeDtypeStruct((V, SD), jnp.float32),
                     jax.ShapeDtypeStruct((V, 1), jnp.float32)]
        out_specs = [pl.BlockSpec((br, SD), lambda h, k: (h * nb + k, 0)),
                     pl.BlockSpec((br, 1), lambda h, k: (h * nb + k, 0))]
        scratch = []
    else:
        in_specs.append(pl.BlockSpec((br, 1), lambda h, k: (h * nb + k, 0)))
        out_shape = [jax.ShapeDtypeStruct((1, 2), jnp.float32)]
        out_specs = [pl.BlockSpec((1, 2), lambda h, k: (0, 0))]
        scratch = _STATS_SCRATCH()

    def body(xs_ref, cnt_ref, *rest):
        h = pl.program_id(0)
        k = pl.program_id(1)
        p = xs_ref[0]
        if first:
            x_out, cnt_out = rest
            cb = cnt_ref[...]
            cnt = cb[0, :, :1] + cb[1, :, :1]
            cnt_out[...] = cnt
            x_out[...] = p * (1.0 / jnp.maximum(cnt, 1.0))
        else:
            st_out, ent_s, col_s, g_s = rest
            x = p * (1.0 / jnp.maximum(cnt_ref[...], 1.0))

            @pl.when((h == 0) & (k == 0))
            def _():
                ent_s[...] = jnp.zeros_like(ent_s)
                col_s[...] = jnp.zeros_like(col_s)
                g_s[...] = jnp.zeros_like(g_s)

            _stats_accumulate(x, ent_s, col_s, g_s)

            @pl.when((h == 1) & (k == nb - 1))
            def _():
                st_out[...] = _stats_final(V, ent_s, col_s, g_s)

    outs = out_shape if len(out_shape) > 1 else out_shape[0]
    return pl.pallas_call(body, grid=(2, nb), in_specs=in_specs,
                          out_specs=out_specs if len(out_shape) > 1 else out_specs[0],
                          out_shape=outs, scratch_shapes=scratch)


def _x_stats_kernel():
    br = 2000
    nb = V // br

    def body(x_ref, st_out, ent_s, col_s, g_s):
        k = pl.program_id(0)
        x = x_ref[...]

        @pl.when(k == 0)
        def _():
            ent_s[...] = jnp.zeros_like(ent_s)
            col_s[...] = jnp.zeros_like(col_s)
            g_s[...] = jnp.zeros_like(g_s)

        _stats_accumulate(x, ent_s, col_s, g_s)

        @pl.when(k == nb - 1)
        def _():
            st_out[...] = _stats_final(V, ent_s, col_s, g_s)

    return pl.pallas_call(
        body, grid=(nb,),
        in_specs=[pl.BlockSpec((br, SD), lambda k: (k, 0))],
        out_specs=pl.BlockSpec((1, 2), lambda k: (0, 0)),
        out_shape=jax.ShapeDtypeStruct((1, 2), jnp.float32),
        scratch_shapes=_STATS_SCRATCH())


def kernel(V_idx, E_idx, node_embedding):
    with jax.ensure_compile_time_eval():
        g = jax.random.gumbel(jax.random.key(42), (V, S, D),
                              dtype=jnp.float32).reshape(V, SD)
    vi = V_idx.astype(jnp.int32)
    ei = E_idx.astype(jnp.int32)
    vidx2 = vi.reshape(NROW, CH)
    eidx2 = ei.reshape(NROW, CH)
    spread = VH + (jnp.arange(NI, dtype=jnp.int32) % 64)
    vs0 = jnp.where(vi < VH, vi, spread).reshape(NROW, CH)
    vs1 = jnp.where(vi >= VH, vi - VH, spread).reshape(NROW, CH)
    vsb = jnp.stack([vs0, vs1])
    z64 = jnp.zeros((XPW, SD), jnp.float32)
    z16 = jnp.zeros((VW, 16), jnp.float32)
    ones16 = jnp.ones((CH, 16), jnp.float32)

    cep, cvp = _run_counts(vidx2, eidx2, z16, ones16)
    z64 = z64 + cep[0, 0, 0] * 0.0
    x = _gumbel_softmax(node_embedding, g)
    locs, globs = [], []
    cnt_e = cnt_v = None
    for step in range(2):
        ysum = _run_y(x, vidx2, eidx2, z64)
        if step == 0:
            y, st, cnt_e = _y_norm_kernel(True)(ysum, cep)
        else:
            y, st = _y_norm_kernel(False)(ysum, cnt_e)
        locs.append(st[0, 0])
        globs.append(st[0, 1])
        xsum = _run_x(y, eidx2, vsb, z64)
        if step == 0:
            x, cnt_v = _x_norm_kernel(True)(xsum, cvp)
            st = _x_stats_kernel()(x)
        else:
            st = _x_norm_kernel(False)(xsum, cnt_v)
        locs.append(st[0, 0])
        globs.append(st[0, 1])
    return jnp.stack([jnp.stack(locs), jnp.stack(globs)])


# speedup vs baseline: 1.0356x; 1.0267x over previous
import jax
import jax.numpy as jnp
from jax import lax
from jax.experimental import pallas as pl
from jax.experimental.pallas import tpu as pltpu
from jax.experimental.pallas import tpu_sc as plsc

EPS = 1e-10
V = 50000
E = 10000
NI = 800000
S = 4
D = 16
SD = S * D
TAU = 1.0

NC = 2
NS = 16
NW = NC * NS

CH = 125
RPB = 8
NROW = NI // CH
TROWS = NROW // NW
NBLK = TROWS // RPB
NBUF = 4
NGRP = TROWS // NBUF

VH = V // 2
XROWS = VH + 88
XPW = XROWS // NS
YW = 1000
VW = 5000

_MESH = dict(core_axis_name="c", subcore_axis_name="s")


def _y_pass_kernel():
    mesh = plsc.VectorSubcoreMesh(**_MESH)
    out_type = jax.ShapeDtypeStruct((NC, E, SD), jnp.float32)
    scratch = ([
        pltpu.VMEM_SHARED((E, SD), jnp.float32),
        pltpu.VMEM((RPB, CH), jnp.int32),
        pltpu.VMEM((RPB, CH), jnp.int32),
    ] + [pltpu.VMEM((CH, SD), jnp.float32) for _ in range(NBUF)]
      + [pltpu.SemaphoreType.DMA for _ in range(NBUF)])

    def body(x_hbm, vidx, eidx, z64, ysum, yacc, vblk, eblk, *rest):
        bufs, sems = rest[:NBUF], rest[NBUF:]
        c = lax.axis_index("c")
        t = lax.axis_index("s")
        base = (t * NC + c) * TROWS

        @pl.when(t < 10)
        def _():
            pltpu.sync_copy(z64.at[pl.ds(0, YW)], yacc.at[pl.ds(t * YW, YW)])
        plsc.subcore_barrier()

        def blk(b, carry):
            r0 = base + b * RPB
            pltpu.sync_copy(vidx.at[pl.ds(r0, RPB)], vblk)
            pltpu.sync_copy(eidx.at[pl.ds(r0, RPB)], eblk)
            for j in range(NBUF):
                pltpu.async_copy(x_hbm.at[vblk.at[j]], bufs[j], sems[j])
            for j in range(RPB):
                jb = j % NBUF
                pltpu.make_async_copy(x_hbm.at[vblk.at[j]], bufs[jb],
                                      sems[jb]).wait()
                pltpu.sync_copy(bufs[jb], yacc.at[eblk.at[j]], add=True)
                if j + NBUF < RPB:
                    pltpu.async_copy(x_hbm.at[vblk.at[j + NBUF]], bufs[jb],
                                     sems[jb])
            return carry
        lax.fori_loop(0, NBLK, blk, 0)
        plsc.subcore_barrier()

        @pl.when(t < 10)
        def _():
            sl = pl.ds(t * YW, YW)
            pltpu.sync_copy(yacc.at[sl], ysum.at[c, sl])

    return pl.kernel(body, out_type=out_type, mesh=mesh, scratch_types=scratch,
                     compiler_params=pltpu.CompilerParams(use_tc_tiling_on_sc=False))


def _counts_kernel():
    mesh = plsc.VectorSubcoreMesh(**_MESH)
    out_type = (jax.ShapeDtypeStruct((NC, E, 16), jnp.float32),
                jax.ShapeDtypeStruct((NC, V, 16), jnp.float32))
    scratch = ([
        pltpu.VMEM_SHARED((E, 16), jnp.float32),
        pltpu.VMEM_SHARED((V, 16), jnp.float32),
        pltpu.VMEM((RPB, CH), jnp.int32),
        pltpu.VMEM((RPB, CH), jnp.int32),
        pltpu.VMEM((CH, 16), jnp.float32),
    ] + [pltpu.SemaphoreType.DMA for _ in range(2 * NBUF)])

    def body(vidx, eidx, z16, ones, ce_out, cv_out, ce, cv, vblk, eblk, onev,
             *sems):
        c = lax.axis_index("c")
        t = lax.axis_index("s")
        base = (t * NC + c) * TROWS
        pltpu.sync_copy(ones, onev)

        @pl.when(t < 10)
        def _():
            pltpu.sync_copy(z16.at[pl.ds(0, YW)], ce.at[pl.ds(t * YW, YW)])
            pltpu.sync_copy(z16, cv.at[pl.ds(t * VW, VW)])
        plsc.subcore_barrier()

        def blk(b, carry):
            r0 = base + b * RPB
            pltpu.sync_copy(vidx.at[pl.ds(r0, RPB)], vblk)
            pltpu.sync_copy(eidx.at[pl.ds(r0, RPB)], eblk)
            for j in range(RPB):
                jb = j % NBUF
                if j >= NBUF:
                    pltpu.make_async_copy(onev, ce.at[eblk.at[j - NBUF]],
                                          sems[jb]).wait()
                    pltpu.make_async_copy(onev, cv.at[vblk.at[j - NBUF]],
                                          sems[NBUF + jb]).wait()
                pltpu.async_copy(onev, ce.at[eblk.at[j]], sems[jb], add=True)
                pltpu.async_copy(onev, cv.at[vblk.at[j]], sems[NBUF + jb],
                                 add=True)
            for j in range(RPB - NBUF, RPB):
                jb = j % NBUF
                pltpu.make_async_copy(onev, ce.at[eblk.at[j]], sems[jb]).wait()
                pltpu.make_async_copy(onev, cv.at[vblk.at[j]],
                                      sems[NBUF + jb]).wait()
            return carry
        lax.fori_loop(0, NBLK, blk, 0)
        plsc.subcore_barrier()

        @pl.when(t < 10)
        def _():
            sl = pl.ds(t * YW, YW)
            pltpu.sync_copy(ce.at[sl], ce_out.at[c, sl])
            slv = pl.ds(t * VW, VW)
            pltpu.sync_copy(cv.at[slv], cv_out.at[c, slv])

    return pl.kernel(body, out_type=out_type, mesh=mesh, scratch_types=scratch,
                     compiler_params=pltpu.CompilerParams(use_tc_tiling_on_sc=False))


def _x_pass_kernel():
    NBX = 3
    TRX = NROW // NS
    NBLKX = TRX // RPB
    mesh = plsc.VectorSubcoreMesh(**_MESH)
    out_type = jax.ShapeDtypeStruct((NC, XROWS, SD), jnp.float32)
    scratch = ([
        pltpu.VMEM_SHARED((XROWS, SD), jnp.float32),
        pltpu.VMEM((RPB, CH), jnp.int32),
        pltpu.VMEM((RPB, CH), jnp.int32),
    ] + [pltpu.VMEM((CH, SD), jnp.float32) for _ in range(NBX)]
      + [pltpu.SemaphoreType.DMA for _ in range(NBX)])

    def body(y_hbm, eidx, vsb, z64, xsum, xacc, eblk, vblk, *rest):
        bufs, sems = rest[:NBX], rest[NBX:]
        c = lax.axis_index("c")
        t = lax.axis_index("s")
        base = t * TRX
        pltpu.sync_copy(z64, xacc.at[pl.ds(t * XPW, XPW)])
        plsc.subcore_barrier()

        def blk(b, carry):
            r0 = base + b * RPB
            pltpu.sync_copy(eidx.at[pl.ds(r0, RPB)], eblk)
            pltpu.sync_copy(vsb.at[c, pl.ds(r0, RPB)], vblk)
            for j in range(NBX):
                pltpu.async_copy(y_hbm.at[eblk.at[j]], bufs[j], sems[j])
            for j in range(RPB):
                jb = j % NBX
                pltpu.make_async_copy(y_hbm.at[eblk.at[j]], bufs[jb],
                                      sems[jb]).wait()
                pltpu.sync_copy(bufs[jb], xacc.at[vblk.at[j]], add=True)
                if j + NBX < RPB:
                    pltpu.async_copy(y_hbm.at[eblk.at[j + NBX]], bufs[jb],
                                     sems[jb])
            return carry
        lax.fori_loop(0, NBLKX, blk, 0)
        plsc.subcore_barrier()

        sl = pl.ds(t * XPW, XPW)
        pltpu.sync_copy(xacc.at[sl], xsum.at[c, sl])

    return pl.kernel(body, out_type=out_type, mesh=mesh, scratch_types=scratch,
                     compiler_params=pltpu.CompilerParams(use_tc_tiling_on_sc=False))


def _run_counts(vidx, eidx, z16, ones):
    return _counts_kernel()(vidx, eidx, z16, ones)


def _run_y(x, vidx, eidx, z64):
    return _y_pass_kernel()(x, vidx, eidx, z64)


def _run_x(y, eidx, vsb, z64):
    return _x_pass_kernel()(y, eidx, vsb, z64)


def _gs_body(emb_ref, g_ref, out_ref):
    x = (emb_ref[...] + g_ref[...]) * (1.0 / TAU)
    ex = jnp.exp(x)
    gi = lax.broadcasted_iota(jnp.int32, (SD, SD), 0) // D
    gj = lax.broadcasted_iota(jnp.int32, (SD, SD), 1) // D
    bd = (gi == gj).astype(jnp.float32)
    s = lax.dot_general(ex, bd, (((1,), (0,)), ((), ())),
                        preferred_element_type=jnp.float32)
    out_ref[...] = ex / s


def _gumbel_softmax(emb, g):
    br = 5000
    return pl.pallas_call(
        _gs_body,
        grid=(V // br,),
        in_specs=[pl.BlockSpec((br, SD), lambda i: (i, 0)),
                  pl.BlockSpec((br, SD), lambda i: (i, 0))],
        out_specs=pl.BlockSpec((br, SD), lambda i: (i, 0)),
        out_shape=jax.ShapeDtypeStruct((V, SD), jnp.float32),
    )(emb, g)


def _stats_accumulate(y, ent_s, col_s, g_s):
    ent_s[...] = ent_s[...] + (-jnp.sum(y * jnp.log(y + EPS)))
    col_s[...] = col_s[...] + jnp.sum(y, axis=0)[None, :]
    g_s[...] = g_s[...] + lax.dot_general(y, y, (((0,), (0,)), ((), ())),
                                          preferred_element_type=jnp.float32)


def _stats_final(n, ent_s, col_s, g_s):
    local = jnp.sum(ent_s[...]) / (n * S)
    pcol = col_s[...] / n
    gb = jnp.sum(pcol * jnp.log(pcol + EPS)) / S
    g = g_s[...]
    eye = (lax.broadcasted_iota(jnp.int32, (D, D), 0) ==
           lax.broadcasted_iota(jnp.int32, (D, D), 1)).astype(jnp.float32)
    disc = jnp.float32(0.0)
    for s in range(S):
        gs = g[s * D:(s + 1) * D, s * D:(s + 1) * D]
        dg = jnp.sum(gs * eye, axis=1)
        norms = jnp.sqrt(dg)
        denom = jnp.maximum(norms[:, None] * norms[None, :], EPS)
        cs = gs / denom
        m = jnp.max(cs, axis=1, keepdims=True)
        ex = jnp.exp(cs - m)
        smd = jnp.sum(ex * eye, axis=1) / jnp.sum(ex, axis=1)
        disc = disc + jnp.sum(-jnp.log(smd))
    disc = disc / (S * D)
    return jnp.stack([local, gb + disc]).reshape(1, 2)


_STATS_SCRATCH = lambda: [pltpu.VMEM((1, 1), jnp.float32),
                          pltpu.VMEM((1, SD), jnp.float32),
                          pltpu.VMEM((SD, SD), jnp.float32)]


def _y_norm_kernel(first):
    br = 2000
    nb = E // br
    in_specs = [pl.BlockSpec((NC, br, SD), lambda k: (0, k, 0))]
    if first:
        in_specs.append(pl.BlockSpec((NC, br, 16), lambda k: (0, k, 0)))
    else:
        in_specs.append(pl.BlockSpec((br, 1), lambda k: (k, 0)))
    out_shape = [jax.ShapeDtypeStruct((E, SD), jnp.float32),
                 jax.ShapeDtypeStruct((1, 2), jnp.float32)]
    out_specs = [pl.BlockSpec((br, SD), lambda k: (k, 0)),
                 pl.BlockSpec((1, 2), lambda k: (0, 0))]
    if first:
        out_shape.append(jax.ShapeDtypeStruct((E, 1), jnp.float32))
        out_specs.append(pl.BlockSpec((br, 1), lambda k: (k, 0)))

    def body(ys_ref, cnt_ref, y_out, st_out, *rest):
        if first:
            cnt_out, ent_s, col_s, g_s = rest
        else:
            ent_s, col_s, g_s = rest
        k = pl.program_id(0)
        a = ys_ref[...]
        p = a[0] + a[1]
        if first:
            cb = cnt_ref[...]
            cnt = cb[0, :, :1] + cb[1, :, :1]
            cnt_out[...] = cnt
        else:
            cnt = cnt_ref[...]
        y = p * (1.0 / jnp.maximum(cnt, 1.0))
        y_out[...] = y

        @pl.when(k == 0)
        def _():
            ent_s[...] = jnp.zeros_like(ent_s)
            col_s[...] = jnp.zeros_like(col_s)
            g_s[...] = jnp.zeros_like(g_s)

        _stats_accumulate(y, ent_s, col_s, g_s)

        @pl.when(k == nb - 1)
        def _():
            st_out[...] = _stats_final(E, ent_s, col_s, g_s)

    return pl.pallas_call(body, grid=(nb,), in_specs=in_specs,
                          out_specs=out_specs, out_shape=out_shape,
                          scratch_shapes=_STATS_SCRATCH())


def _x_norm_kernel(first):
    br = 5000
    nb = VH // br
    in_specs = [pl.BlockSpec((1, br, SD), lambda h, k: (h, k, 0))]
    if first:
        in_specs.append(pl.BlockSpec((NC, br, 16), lambda h, k: (0, h * nb + k, 0)))
        out_shape = [jax.ShapeDtypeStruct((V, SD), jnp.float32),
                     jax.ShapeDtypeStruct((V, 1), jnp.float32)]
        out_specs = [pl.BlockSpec((br, SD), lambda h, k: (h * nb + k, 0)),
                     pl.BlockSpec((br, 1), lambda h, k: (h * nb + k, 0))]
        scratch = []
    else:
        in_specs.append(pl.BlockSpec((br, 1), lambda h, k: (h * nb + k, 0)))
        out_shape = [jax.ShapeDtypeStruct((1, 2), jnp.float32)]
        out_specs = [pl.BlockSpec((1, 2), lambda h, k: (0, 0))]
        scratch = _STATS_SCRATCH()

    def body(xs_ref, cnt_ref, *rest):
        h = pl.program_id(0)
        k = pl.program_id(1)
        p = xs_ref[0]
        if first:
            x_out, cnt_out = rest
            cb = cnt_ref[...]
            cnt = cb[0, :, :1] + cb[1, :, :1]
            cnt_out[...] = cnt
            x_out[...] = p * (1.0 / jnp.maximum(cnt, 1.0))
        else:
            st_out, ent_s, col_s, g_s = rest
            x = p * (1.0 / jnp.maximum(cnt_ref[...], 1.0))

            @pl.when((h == 0) & (k == 0))
            def _():
                ent_s[...] = jnp.zeros_like(ent_s)
                col_s[...] = jnp.zeros_like(col_s)
                g_s[...] = jnp.zeros_like(g_s)

            _stats_accumulate(x, ent_s, col_s, g_s)

            @pl.when((h == 1) & (k == nb - 1))
            def _():
                st_out[...] = _stats_final(V, ent_s, col_s, g_s)

    outs = out_shape if len(out_shape) > 1 else out_shape[0]
    return pl.pallas_call(body, grid=(2, nb), in_specs=in_specs,
                          out_specs=out_specs if len(out_shape) > 1 else out_specs[0],
                          out_shape=outs, scratch_shapes=scratch)


def _x_stats_kernel():
    br = 2000
    nb = V // br

    def body(x_ref, st_out, ent_s, col_s, g_s):
        k = pl.program_id(0)
        x = x_ref[...]

        @pl.when(k == 0)
        def _():
            ent_s[...] = jnp.zeros_like(ent_s)
            col_s[...] = jnp.zeros_like(col_s)
            g_s[...] = jnp.zeros_like(g_s)

        _stats_accumulate(x, ent_s, col_s, g_s)

        @pl.when(k == nb - 1)
        def _():
            st_out[...] = _stats_final(V, ent_s, col_s, g_s)

    return pl.pallas_call(
        body, grid=(nb,),
        in_specs=[pl.BlockSpec((br, SD), lambda k: (k, 0))],
        out_specs=pl.BlockSpec((1, 2), lambda k: (0, 0)),
        out_shape=jax.ShapeDtypeStruct((1, 2), jnp.float32),
        scratch_shapes=_STATS_SCRATCH())


def kernel(V_idx, E_idx, node_embedding):
    with jax.ensure_compile_time_eval():
        g = jax.random.gumbel(jax.random.key(42), (V, S, D),
                              dtype=jnp.float32).reshape(V, SD)
    vi = V_idx.astype(jnp.int32)
    ei = E_idx.astype(jnp.int32)
    vidx2 = vi.reshape(NROW, CH)
    eidx2 = ei.reshape(NROW, CH)
    spread = VH + (jnp.arange(NI, dtype=jnp.int32) % 64)
    vs0 = jnp.where(vi < VH, vi, spread).reshape(NROW, CH)
    vs1 = jnp.where(vi >= VH, vi - VH, spread).reshape(NROW, CH)
    vsb = jnp.stack([vs0, vs1])
    z64 = jnp.zeros((XPW, SD), jnp.float32)
    z16 = jnp.zeros((VW, 16), jnp.float32)
    ones16 = jnp.ones((CH, 16), jnp.float32)

    cep, cvp = _run_counts(vidx2, eidx2, z16, ones16)
    z64 = z64 + cep[0, 0, 0] * 0.0
    x = _gumbel_softmax(node_embedding, g)
    locs, globs = [], []
    cnt_e = cnt_v = None
    for step in range(2):
        ysum = _run_y(x, vidx2, eidx2, z64)
        if step == 0:
            y, st, cnt_e = _y_norm_kernel(True)(ysum, cep)
        else:
            y, st = _y_norm_kernel(False)(ysum, cnt_e)
        locs.append(st[0, 0])
        globs.append(st[0, 1])
        xsum = _run_x(y, eidx2, vsb, z64)
        if step == 0:
            x, cnt_v = _x_norm_kernel(True)(xsum, cvp)
            st = _x_stats_kernel()(x)
        else:
            st = _x_norm_kernel(False)(xsum, cnt_v)
        locs.append(st[0, 0])
        globs.append(st[0, 1])
    return jnp.stack([jnp.stack(locs), jnp.stack(globs)])


# speedup vs baseline: 1.0380x; 1.0023x over previous
import jax
import jax.numpy as jnp
from jax import lax
from jax.experimental import pallas as pl
from jax.experimental.pallas import tpu as pltpu
from jax.experimental.pallas import tpu_sc as plsc

EPS = 1e-10
V = 50000
E = 10000
NI = 800000
S = 4
D = 16
SD = S * D
TAU = 1.0

NC = 2
NS = 16
NW = NC * NS

CH = 125
RPB = 8
NROW = NI // CH
TROWS = NROW // NW
NBLK = TROWS // RPB
NBUF = 4
NGRP = TROWS // NBUF

VH = V // 2
XROWS = VH + 88
XPW = XROWS // NS
YW = 1000
VW = 5000

_MESH = dict(core_axis_name="c", subcore_axis_name="s")


def _y_pass_kernel():
    mesh = plsc.VectorSubcoreMesh(**_MESH)
    out_type = jax.ShapeDtypeStruct((NC, E, SD), jnp.float32)
    scratch = ([
        pltpu.VMEM_SHARED((E, SD), jnp.float32),
        pltpu.VMEM((RPB, CH), jnp.int32),
        pltpu.VMEM((RPB, CH), jnp.int32),
    ] + [pltpu.VMEM((CH, SD), jnp.float32) for _ in range(NBUF)]
      + [pltpu.SemaphoreType.DMA for _ in range(NBUF)])

    def body(x_hbm, vidx, eidx, z64, ysum, yacc, vblk, eblk, *rest):
        bufs, sems = rest[:NBUF], rest[NBUF:]
        c = lax.axis_index("c")
        t = lax.axis_index("s")
        base = (t * NC + c) * TROWS

        @pl.when(t < 10)
        def _():
            pltpu.sync_copy(z64.at[pl.ds(0, YW)], yacc.at[pl.ds(t * YW, YW)])
        plsc.subcore_barrier()

        def blk(b, carry):
            r0 = base + b * RPB
            pltpu.sync_copy(vidx.at[pl.ds(r0, RPB)], vblk)
            pltpu.sync_copy(eidx.at[pl.ds(r0, RPB)], eblk)
            for j in range(NBUF):
                pltpu.async_copy(x_hbm.at[vblk.at[j]], bufs[j], sems[j])
            for j in range(RPB):
                jb = j % NBUF
                pltpu.make_async_copy(x_hbm.at[vblk.at[j]], bufs[jb],
                                      sems[jb]).wait()
                pltpu.sync_copy(bufs[jb], yacc.at[eblk.at[j]], add=True)
                if j + NBUF < RPB:
                    pltpu.async_copy(x_hbm.at[vblk.at[j + NBUF]], bufs[jb],
                                     sems[jb])
            return carry
        lax.fori_loop(0, NBLK, blk, 0)
        plsc.subcore_barrier()

        @pl.when(t < 10)
        def _():
            sl = pl.ds(t * YW, YW)
            pltpu.sync_copy(yacc.at[sl], ysum.at[c, sl])

    return pl.kernel(body, out_type=out_type, mesh=mesh, scratch_types=scratch,
                     compiler_params=pltpu.CompilerParams(use_tc_tiling_on_sc=False))


def _counts_kernel():
    mesh = plsc.VectorSubcoreMesh(**_MESH)
    out_type = (jax.ShapeDtypeStruct((NC, E, 16), jnp.float32),
                jax.ShapeDtypeStruct((NC, V, 16), jnp.float32))
    scratch = ([
        pltpu.VMEM_SHARED((E, 16), jnp.float32),
        pltpu.VMEM_SHARED((V, 16), jnp.float32),
        pltpu.VMEM((RPB, CH), jnp.int32),
        pltpu.VMEM((RPB, CH), jnp.int32),
        pltpu.VMEM((CH, 16), jnp.float32),
    ] + [pltpu.SemaphoreType.DMA for _ in range(2 * NBUF)])

    def body(vidx, eidx, z16, ones, ce_out, cv_out, ce, cv, vblk, eblk, onev,
             *sems):
        c = lax.axis_index("c")
        t = lax.axis_index("s")
        base = (t * NC + c) * TROWS
        pltpu.sync_copy(ones, onev)

        @pl.when(t < 10)
        def _():
            pltpu.sync_copy(z16.at[pl.ds(0, YW)], ce.at[pl.ds(t * YW, YW)])
            pltpu.sync_copy(z16, cv.at[pl.ds(t * VW, VW)])
        plsc.subcore_barrier()

        def blk(b, carry):
            r0 = base + b * RPB
            pltpu.sync_copy(vidx.at[pl.ds(r0, RPB)], vblk)
            pltpu.sync_copy(eidx.at[pl.ds(r0, RPB)], eblk)
            for j in range(RPB):
                jb = j % NBUF
                if j >= NBUF:
                    pltpu.make_async_copy(onev, ce.at[eblk.at[j - NBUF]],
                                          sems[jb]).wait()
                    pltpu.make_async_copy(onev, cv.at[vblk.at[j - NBUF]],
                                          sems[NBUF + jb]).wait()
                pltpu.async_copy(onev, ce.at[eblk.at[j]], sems[jb], add=True)
                pltpu.async_copy(onev, cv.at[vblk.at[j]], sems[NBUF + jb],
                                 add=True)
            for j in range(RPB - NBUF, RPB):
                jb = j % NBUF
                pltpu.make_async_copy(onev, ce.at[eblk.at[j]], sems[jb]).wait()
                pltpu.make_async_copy(onev, cv.at[vblk.at[j]],
                                      sems[NBUF + jb]).wait()
            return carry
        lax.fori_loop(0, NBLK, blk, 0)
        plsc.subcore_barrier()

        @pl.when(t < 10)
        def _():
            sl = pl.ds(t * YW, YW)
            pltpu.sync_copy(ce.at[sl], ce_out.at[c, sl])
            slv = pl.ds(t * VW, VW)
            pltpu.sync_copy(cv.at[slv], cv_out.at[c, slv])

    return pl.kernel(body, out_type=out_type, mesh=mesh, scratch_types=scratch,
                     compiler_params=pltpu.CompilerParams(use_tc_tiling_on_sc=False))


def _x_pass_kernel():
    NBX = 3
    TRX = NROW // NS
    NBLKX = TRX // RPB
    mesh = plsc.VectorSubcoreMesh(**_MESH)
    out_type = jax.ShapeDtypeStruct((NC, XROWS, SD), jnp.float32)
    scratch = ([
        pltpu.VMEM_SHARED((XROWS, SD), jnp.float32),
        pltpu.VMEM((RPB, CH), jnp.int32),
        pltpu.VMEM((RPB, CH), jnp.int32),
    ] + [pltpu.VMEM((CH, SD), jnp.float32) for _ in range(NBX)]
      + [pltpu.SemaphoreType.DMA for _ in range(NBX)])

    def body(y_hbm, eidx, vsb, z64, xsum, xacc, eblk, vblk, *rest):
        bufs, sems = rest[:NBX], rest[NBX:]
        c = lax.axis_index("c")
        t = lax.axis_index("s")
        base = t * TRX
        pltpu.sync_copy(z64, xacc.at[pl.ds(t * XPW, XPW)])
        plsc.subcore_barrier()

        def blk(b, carry):
            r0 = base + b * RPB
            pltpu.sync_copy(eidx.at[pl.ds(r0, RPB)], eblk)
            pltpu.sync_copy(vsb.at[c, pl.ds(r0, RPB)], vblk)
            for j in range(NBX):
                pltpu.async_copy(y_hbm.at[eblk.at[j]], bufs[j], sems[j])
            for j in range(RPB):
                jb = j % NBX
                pltpu.make_async_copy(y_hbm.at[eblk.at[j]], bufs[jb],
                                      sems[jb]).wait()
                pltpu.sync_copy(bufs[jb], xacc.at[vblk.at[j]], add=True)
                if j + NBX < RPB:
                    pltpu.async_copy(y_hbm.at[eblk.at[j + NBX]], bufs[jb],
                                     sems[jb])
            return carry
        lax.fori_loop(0, NBLKX, blk, 0)
        plsc.subcore_barrier()

        sl = pl.ds(t * XPW, XPW)
        pltpu.sync_copy(xacc.at[sl], xsum.at[c, sl])

    return pl.kernel(body, out_type=out_type, mesh=mesh, scratch_types=scratch,
                     compiler_params=pltpu.CompilerParams(use_tc_tiling_on_sc=False))


def _run_counts(vidx, eidx, z16, ones):
    return _counts_kernel()(vidx, eidx, z16, ones)


def _run_y(x, vidx, eidx, z64):
    return _y_pass_kernel()(x, vidx, eidx, z64)


def _run_x(y, eidx, vsb, z64):
    return _x_pass_kernel()(y, eidx, vsb, z64)


def _gs_body(emb_ref, g_ref, out_ref):
    x = (emb_ref[...] + g_ref[...]) * (1.0 / TAU)
    ex = jnp.exp(x)
    gi = lax.broadcasted_iota(jnp.int32, (SD, SD), 0) // D
    gj = lax.broadcasted_iota(jnp.int32, (SD, SD), 1) // D
    bd = (gi == gj).astype(jnp.float32)
    s = lax.dot_general(ex, bd, (((1,), (0,)), ((), ())),
                        preferred_element_type=jnp.float32)
    out_ref[...] = ex / s


def _gumbel_softmax(emb, g):
    br = 10000
    return pl.pallas_call(
        _gs_body,
        grid=(V // br,),
        in_specs=[pl.BlockSpec((br, SD), lambda i: (i, 0)),
                  pl.BlockSpec((br, SD), lambda i: (i, 0))],
        out_specs=pl.BlockSpec((br, SD), lambda i: (i, 0)),
        out_shape=jax.ShapeDtypeStruct((V, SD), jnp.float32),
    )(emb, g)


def _stats_accumulate(y, ent_s, col_s, g_s):
    ent_s[...] = ent_s[...] + (-jnp.sum(y * jnp.log(y + EPS)))
    col_s[...] = col_s[...] + jnp.sum(y, axis=0)[None, :]
    g_s[...] = g_s[...] + lax.dot_general(y, y, (((0,), (0,)), ((), ())),
                                          preferred_element_type=jnp.float32)


def _stats_final(n, ent_s, col_s, g_s):
    local = jnp.sum(ent_s[...]) / (n * S)
    pcol = col_s[...] / n
    gb = jnp.sum(pcol * jnp.log(pcol + EPS)) / S
    g = g_s[...]
    eye = (lax.broadcasted_iota(jnp.int32, (D, D), 0) ==
           lax.broadcasted_iota(jnp.int32, (D, D), 1)).astype(jnp.float32)
    disc = jnp.float32(0.0)
    for s in range(S):
        gs = g[s * D:(s + 1) * D, s * D:(s + 1) * D]
        dg = jnp.sum(gs * eye, axis=1)
        norms = jnp.sqrt(dg)
        denom = jnp.maximum(norms[:, None] * norms[None, :], EPS)
        cs = gs / denom
        m = jnp.max(cs, axis=1, keepdims=True)
        ex = jnp.exp(cs - m)
        smd = jnp.sum(ex * eye, axis=1) / jnp.sum(ex, axis=1)
        disc = disc + jnp.sum(-jnp.log(smd))
    disc = disc / (S * D)
    return jnp.stack([local, gb + disc]).reshape(1, 2)


_STATS_SCRATCH = lambda: [pltpu.VMEM((1, 1), jnp.float32),
                          pltpu.VMEM((1, SD), jnp.float32),
                          pltpu.VMEM((SD, SD), jnp.float32)]


def _y_norm_kernel(first):
    br = 10000
    nb = E // br
    in_specs = [pl.BlockSpec((NC, br, SD), lambda k: (0, k, 0))]
    if first:
        in_specs.append(pl.BlockSpec((NC, br, 16), lambda k: (0, k, 0)))
    else:
        in_specs.append(pl.BlockSpec((br, 1), lambda k: (k, 0)))
    out_shape = [jax.ShapeDtypeStruct((E, SD), jnp.float32),
                 jax.ShapeDtypeStruct((1, 2), jnp.float32)]
    out_specs = [pl.BlockSpec((br, SD), lambda k: (k, 0)),
                 pl.BlockSpec((1, 2), lambda k: (0, 0))]
    if first:
        out_shape.append(jax.ShapeDtypeStruct((E, 1), jnp.float32))
        out_specs.append(pl.BlockSpec((br, 1), lambda k: (k, 0)))

    def body(ys_ref, cnt_ref, y_out, st_out, *rest):
        if first:
            cnt_out, ent_s, col_s, g_s = rest
        else:
            ent_s, col_s, g_s = rest
        k = pl.program_id(0)
        a = ys_ref[...]
        p = a[0] + a[1]
        if first:
            cb = cnt_ref[...]
            cnt = cb[0, :, :1] + cb[1, :, :1]
            cnt_out[...] = cnt
        else:
            cnt = cnt_ref[...]
        y = p * (1.0 / jnp.maximum(cnt, 1.0))
        y_out[...] = y

        @pl.when(k == 0)
        def _():
            ent_s[...] = jnp.zeros_like(ent_s)
            col_s[...] = jnp.zeros_like(col_s)
            g_s[...] = jnp.zeros_like(g_s)

        _stats_accumulate(y, ent_s, col_s, g_s)

        @pl.when(k == nb - 1)
        def _():
            st_out[...] = _stats_final(E, ent_s, col_s, g_s)

    return pl.pallas_call(body, grid=(nb,), in_specs=in_specs,
                          out_specs=out_specs, out_shape=out_shape,
                          scratch_shapes=_STATS_SCRATCH())


def _x_norm_kernel(first):
    br = 5000
    nb = VH // br
    in_specs = [pl.BlockSpec((1, br, SD), lambda h, k: (h, k, 0))]
    if first:
        in_specs.append(pl.BlockSpec((NC, br, 16), lambda h, k: (0, h * nb + k, 0)))
        out_shape = [jax.ShapeDtypeStruct((V, SD), jnp.float32),
                     jax.ShapeDtypeStruct((V, 1), jnp.float32)]
        out_specs = [pl.BlockSpec((br, SD), lambda h, k: (h * nb + k, 0)),
                     pl.BlockSpec((br, 1), lambda h, k: (h * nb + k, 0))]
        scratch = []
    else:
        in_specs.append(pl.BlockSpec((br, 1), lambda h, k: (h * nb + k, 0)))
        out_shape = [jax.ShapeDtypeStruct((1, 2), jnp.float32)]
        out_specs = [pl.BlockSpec((1, 2), lambda h, k: (0, 0))]
        scratch = _STATS_SCRATCH()

    def body(xs_ref, cnt_ref, *rest):
        h = pl.program_id(0)
        k = pl.program_id(1)
        p = xs_ref[0]
        if first:
            x_out, cnt_out = rest
            cb = cnt_ref[...]
            cnt = cb[0, :, :1] + cb[1, :, :1]
            cnt_out[...] = cnt
            x_out[...] = p * (1.0 / jnp.maximum(cnt, 1.0))
        else:
            st_out, ent_s, col_s, g_s = rest
            x = p * (1.0 / jnp.maximum(cnt_ref[...], 1.0))

            @pl.when((h == 0) & (k == 0))
            def _():
                ent_s[...] = jnp.zeros_like(ent_s)
                col_s[...] = jnp.zeros_like(col_s)
                g_s[...] = jnp.zeros_like(g_s)

            _stats_accumulate(x, ent_s, col_s, g_s)

            @pl.when((h == 1) & (k == nb - 1))
            def _():
                st_out[...] = _stats_final(V, ent_s, col_s, g_s)

    outs = out_shape if len(out_shape) > 1 else out_shape[0]
    return pl.pallas_call(body, grid=(2, nb), in_specs=in_specs,
                          out_specs=out_specs if len(out_shape) > 1 else out_specs[0],
                          out_shape=outs, scratch_shapes=scratch)


def _x_stats_kernel():
    br = 2000
    nb = V // br

    def body(x_ref, st_out, ent_s, col_s, g_s):
        k = pl.program_id(0)
        x = x_ref[...]

        @pl.when(k == 0)
        def _():
            ent_s[...] = jnp.zeros_like(ent_s)
            col_s[...] = jnp.zeros_like(col_s)
            g_s[...] = jnp.zeros_like(g_s)

        _stats_accumulate(x, ent_s, col_s, g_s)

        @pl.when(k == nb - 1)
        def _():
            st_out[...] = _stats_final(V, ent_s, col_s, g_s)

    return pl.pallas_call(
        body, grid=(nb,),
        in_specs=[pl.BlockSpec((br, SD), lambda k: (k, 0))],
        out_specs=pl.BlockSpec((1, 2), lambda k: (0, 0)),
        out_shape=jax.ShapeDtypeStruct((1, 2), jnp.float32),
        scratch_shapes=_STATS_SCRATCH())


def kernel(V_idx, E_idx, node_embedding):
    with jax.ensure_compile_time_eval():
        g = jax.random.gumbel(jax.random.key(42), (V, S, D),
                              dtype=jnp.float32).reshape(V, SD)
    vi = V_idx.astype(jnp.int32)
    ei = E_idx.astype(jnp.int32)
    vidx2 = vi.reshape(NROW, CH)
    eidx2 = ei.reshape(NROW, CH)
    spread = VH + (jnp.arange(NI, dtype=jnp.int32) % 64)
    vs0 = jnp.where(vi < VH, vi, spread).reshape(NROW, CH)
    vs1 = jnp.where(vi >= VH, vi - VH, spread).reshape(NROW, CH)
    vsb = jnp.stack([vs0, vs1])
    z64 = jnp.zeros((XPW, SD), jnp.float32)
    z16 = jnp.zeros((VW, 16), jnp.float32)
    ones16 = jnp.ones((CH, 16), jnp.float32)

    cep, cvp = _run_counts(vidx2, eidx2, z16, ones16)
    z64 = z64 + cep[0, 0, 0] * 0.0
    x = _gumbel_softmax(node_embedding, g)
    locs, globs = [], []
    cnt_e = cnt_v = None
    for step in range(2):
        ysum = _run_y(x, vidx2, eidx2, z64)
        if step == 0:
            y, st, cnt_e = _y_norm_kernel(True)(ysum, cep)
        else:
            y, st = _y_norm_kernel(False)(ysum, cnt_e)
        locs.append(st[0, 0])
        globs.append(st[0, 1])
        xsum = _run_x(y, eidx2, vsb, z64)
        if step == 0:
            x, cnt_v = _x_norm_kernel(True)(xsum, cvp)
            st = _x_stats_kernel()(x)
        else:
            st = _x_norm_kernel(False)(xsum, cnt_v)
        locs.append(st[0, 0])
        globs.append(st[0, 1])
    return jnp.stack([jnp.stack(locs), jnp.stack(globs)])
